# baseline (speedup 1.0000x reference)
import functools

import jax
import jax.numpy as jnp
from jax import lax
from jax.experimental import pallas as pl
from jax.experimental.pallas import tpu as pltpu
from jax.experimental.pallas import tpu_sc as plsc


@functools.cache
def _build(B, F, V, D):
    info = plsc.get_sparse_core_info()
    NC, NS, L = info.num_cores, info.num_subcores, info.num_lanes
    NW = NC * NS
    assert D == NW
    SP = 8
    BH = B // 2
    mesh = plsc.VectorSubcoreMesh(core_axis_name="c", subcore_axis_name="s")

    @functools.partial(
        pl.kernel,
        mesh=mesh,
        out_type=jax.ShapeDtypeStruct((F, D, B), jnp.float32),
        scratch_types=(
            [pltpu.VMEM_SHARED((V,), jnp.float32) for _ in range(2 * SP)]
            + [
                pltpu.VMEM((BH,), jnp.int32),
                pltpu.VMEM((BH,), jnp.float32),
                pltpu.VMEM((D,), jnp.float32),
                pltpu.SemaphoreType.DMA,
                pltpu.SemaphoreType.DMA,
            ]
        ),
        compiler_params=pltpu.CompilerParams(needs_layout_passes=False),
    )
    def gather_bias(tab_t, x_t, col, out_t, *refs):
        banks = (refs[0:SP], refs[SP:2 * SP])
        idx_v, out_v, col_v, sem_g, sem_st = refs[2 * SP:]
        c = lax.axis_index("c")
        s = lax.axis_index("s")
        p = s % SP
        h = s // SP
        base_d = c * (D // NC)

        def stage(bank, f_n, g_n, guard, start):
            for ss in range(SP):
                @pl.when(jnp.logical_and(s == ss, guard))
                def _():
                    cp = pltpu.async_copy(
                        tab_t.at[f_n, base_d + g_n * SP + ss],
                        bank[ss], sem_st)
                    if not start:
                        cp.wait()
            return None

        def stage_wait(bank, f_n, g_n, guard):
            for ss in range(SP):
                @pl.when(jnp.logical_and(s == ss, guard))
                def _():
                    pltpu.make_async_copy(
                        tab_t.at[f_n, base_d + g_n * SP + ss],
                        bank[ss], sem_st).wait()

        for ss in range(SP):
            @pl.when(s == ss)
            def _():
                pltpu.sync_copy(tab_t.at[0, base_d + ss], banks[0][ss])
        plsc.subcore_barrier()

        def field_body(f, carry):
            for g in range(2):
                cur = banks[g]
                oth = banks[1 - g]
                f1 = f if g == 0 else f + 1
                g1 = 1 - g
                guard = (f1 < F) if g == 1 else (f >= 0)
                stage(oth, f1, g1, guard, start=True)

                d = base_d + g * SP + p
                pltpu.sync_copy(x_t.at[f, pl.ds(h * BH, BH)], idx_v)
                pltpu.sync_copy(col.at[f], col_v)
                bias = plsc.load_gather(
                    col_v, [jnp.full((L,), d, jnp.int32)])
                for pp in range(SP):
                    @pl.when(p == pp)
                    def _():
                        pltpu.async_copy(
                            cur[pp].at[idx_v], out_v, sem_g).wait()

                @plsc.parallel_loop(0, BH // L, unroll=8)
                def ibody(i):
                    out_v[pl.ds(i * L, L)] = out_v[pl.ds(i * L, L)] + bias

                pltpu.sync_copy(out_v, out_t.at[f, d, pl.ds(h * BH, BH)])
                stage_wait(oth, f1, g1, guard)
                plsc.subcore_barrier()
            return carry

        lax.fori_loop(0, F, field_body, 0)

    return gather_bias


def kernel(x_cat, tables, col_embed):
    F, V, D = tables.shape
    B = x_cat.shape[0]
    tab_t = tables.transpose(0, 2, 1)
    x_t = x_cat.astype(jnp.int32).T
    out_t = _build(B, F, V, D)(tab_t, x_t, col_embed)
    return out_t.transpose(2, 0, 1)


# speedup vs baseline: 1.0070x; 1.0070x over previous
import functools

import jax
import jax.numpy as jnp
from jax import lax
from jax.experimental import pallas as pl
from jax.experimental.pallas import tpu as pltpu
from jax.experimental.pallas import tpu_sc as plsc


@functools.cache
def _build(B, F, V, D):
    info = plsc.get_sparse_core_info()
    NC, NS, L = info.num_cores, info.num_subcores, info.num_lanes
    NW = NC * NS
    assert D == NW, "one emb_dim plane per vector subcore"
    S = (V // 2) // 128 * 128
    HI = (V - S) // 128 * 128
    TAIL = V - S - HI
    assert 0 < TAIL <= L * 2 and TAIL % L == 0
    CHO = 2048
    NCH = B // CHO
    mesh = plsc.VectorSubcoreMesh(core_axis_name="c", subcore_axis_name="s")

    @functools.partial(
        pl.kernel,
        mesh=mesh,
        out_type=jax.ShapeDtypeStruct((F, D, B), jnp.float32),
        scratch_types=[
            pltpu.VMEM((S,), jnp.float32),
            pltpu.VMEM((HI + TAIL,), jnp.float32),
            pltpu.VMEM((TAIL,), jnp.float32),
            pltpu.VMEM((2, CHO), jnp.int32),
            pltpu.VMEM((B,), jnp.float32),
            pltpu.VMEM((D,), jnp.float32),
            pltpu.SemaphoreType.DMA,
            pltpu.SemaphoreType.DMA,
            pltpu.SemaphoreType.DMA,
            pltpu.SemaphoreType.DMA,
            pltpu.SemaphoreType.DMA,
            pltpu.SemaphoreType.DMA,
        ],
        compiler_params=pltpu.CompilerParams(needs_layout_passes=False),
    )
    def gather_bias(tab_t, x_t, col, tail3, out_t, lo_v, hi_v, tail_v,
                    idx_v, out_v, col_v, sem_a, sem_b, sem_t, sem_i0,
                    sem_i1, sem_o):
        w = lax.axis_index("s") * NC + lax.axis_index("c")
        w16 = jnp.full((L,), w, jnp.int32)
        sem_i = (sem_i0, sem_i1)

        def lo_cp(f):
            return pltpu.make_async_copy(
                tab_t.at[f, w].at[pl.ds(0, S)], lo_v, sem_a)

        def hi_cp(f):
            return pltpu.make_async_copy(
                tab_t.at[f, w].at[pl.ds(S, HI)],
                hi_v.at[pl.ds(0, HI)], sem_b)

        def tail_cp(f):
            return pltpu.make_async_copy(tail3.at[f, w], tail_v, sem_t)

        def idx_cp(f, c, b):
            return pltpu.make_async_copy(
                x_t.at[f, pl.ds(c * CHO, CHO)], idx_v.at[b], sem_i[b])

        def out_cp(f, c):
            return pltpu.make_async_copy(
                out_v.at[pl.ds(c * CHO, CHO)],
                out_t.at[f, w, pl.ds(c * CHO, CHO)], sem_o)

        lo_cp(0).start()

        def field_body(f, carry):
            lo_cp(f).wait()
            hi_cp(f).start()
            tail_cp(f).start()
            pltpu.sync_copy(col.at[f], col_v)
            bias = plsc.load_gather(col_v, [w16])

            idx_cp(f, 0, 0).start()
            for c in range(NCH):
                b0 = c % 2
                idx_cp(f, c, b0).wait()
                if c + 1 < NCH:
                    idx_cp(f, c + 1, (c + 1) % 2).start()

                @plsc.parallel_loop(0, CHO // L, unroll=8)
                def p0body(i):
                    idx16 = idx_v[b0, pl.ds(i * L, L)]
                    m0 = idx16 < S
                    g0 = plsc.load_gather(
                        lo_v, [jnp.minimum(idx16, S - 1)], mask=m0)
                    out_v[pl.ds(c * CHO + i * L, L)] = g0 + bias

            hi_cp(f).wait()
            tail_cp(f).wait()
            for t in range(TAIL // L):
                hi_v[pl.ds(HI + t * L, L)] = tail_v[pl.ds(t * L, L)]

            @pl.when(f + 1 < F)
            def _():
                lo_cp(f + 1).start()

            idx_cp(f, 0, 0).start()
            for c in range(NCH):
                b0 = c % 2
                idx_cp(f, c, b0).wait()
                if c + 1 < NCH:
                    idx_cp(f, c + 1, (c + 1) % 2).start()

                @plsc.parallel_loop(0, CHO // L, unroll=8)
                def p1body(i):
                    idx16 = idx_v[b0, pl.ds(i * L, L)]
                    m1 = idx16 >= S
                    g1 = plsc.load_gather(
                        hi_v, [jnp.maximum(idx16 - S, 0)], mask=m1)
                    cur = out_v[pl.ds(c * CHO + i * L, L)]
                    out_v[pl.ds(c * CHO + i * L, L)] = jnp.where(
                        m1, g1 + bias, cur)

                out_cp(f, c).start()
            for c in range(NCH):
                out_cp(f, c).wait()
            return carry

        lax.fori_loop(0, F, field_body, 0)

    return gather_bias


def kernel(x_cat, tables, col_embed):
    F, V, D = tables.shape
    B = x_cat.shape[0]
    tab_t = tables.transpose(0, 2, 1)
    x_t = x_cat.astype(jnp.int32).T
    S = (V // 2) // 128 * 128
    HI = (V - S) // 128 * 128
    tail3 = tab_t[:, :, S + HI:]
    out_t = _build(B, F, V, D)(tab_t, x_t, col_embed, tail3)
    return out_t.transpose(2, 0, 1)


# speedup vs baseline: 1.2513x; 1.2425x over previous
import functools

import jax
import jax.numpy as jnp
from jax import lax
from jax.experimental import pallas as pl
from jax.experimental.pallas import tpu as pltpu
from jax.experimental.pallas import tpu_sc as plsc


@functools.cache
def _build(B, F, V, D):
    info = plsc.get_sparse_core_info()
    NC, NS, L = info.num_cores, info.num_subcores, info.num_lanes
    NW = NC * NS
    assert D == NW, "one emb_dim plane per vector subcore"
    CHO = 2048
    NCH = B // CHO
    assert B % CHO == 0 and CHO % L == 0
    mesh = plsc.VectorSubcoreMesh(core_axis_name="c", subcore_axis_name="s")

    @functools.partial(
        pl.kernel,
        mesh=mesh,
        out_type=jax.ShapeDtypeStruct((F, D, B), jnp.float32),
        scratch_types=[
            pltpu.VMEM((V,), jnp.float32),
            pltpu.VMEM((2, CHO), jnp.int32),
            pltpu.VMEM((2, CHO), jnp.float32),
            pltpu.VMEM((D,), jnp.float32),
            pltpu.SemaphoreType.DMA,
            pltpu.SemaphoreType.DMA,
            pltpu.SemaphoreType.DMA,
            pltpu.SemaphoreType.DMA,
        ],
        compiler_params=pltpu.CompilerParams(needs_layout_passes=False, use_tc_tiling_on_sc=True),
    )
    def gather_bias(tab_t, x_t, col, out_t, plane_v, idx_v, out_v, col_v,
                    sem_i0, sem_i1, sem_o0, sem_o1):
        w = lax.axis_index("s") * NC + lax.axis_index("c")
        w16 = jnp.full((L,), w, jnp.int32)
        sem_i = (sem_i0, sem_i1)
        sem_o = (sem_o0, sem_o1)

        def field_body(f, carry):
            pltpu.sync_copy(tab_t.at[f, w], plane_v)
            pltpu.sync_copy(col.at[f], col_v)
            bias = plsc.load_gather(col_v, [w16])
            idx_cp = [None, None]
            out_cp = [None, None]
            idx_cp[0] = pltpu.async_copy(
                x_t.at[f, pl.ds(0, CHO)], idx_v.at[0], sem_i[0])
            for c in range(NCH):
                b0 = c % 2
                idx_cp[b0].wait()
                if c + 1 < NCH:
                    b1 = (c + 1) % 2
                    idx_cp[b1] = pltpu.async_copy(
                        x_t.at[f, pl.ds((c + 1) * CHO, CHO)],
                        idx_v.at[b1], sem_i[b1])
                if out_cp[b0] is not None:
                    out_cp[b0].wait()

                @plsc.parallel_loop(0, CHO // L, unroll=8)
                def ibody(i):
                    idx16 = idx_v[b0, pl.ds(i * L, L)]
                    out_v[b0, pl.ds(i * L, L)] = (
                        plsc.load_gather(plane_v, [idx16]) + bias
                    )

                out_cp[b0] = pltpu.async_copy(
                    out_v.at[b0], out_t.at[f, w, pl.ds(c * CHO, CHO)],
                    sem_o[b0])
            out_cp[0].wait()
            out_cp[1].wait()
            return carry

        lax.fori_loop(0, F, field_body, 0)

    return gather_bias


def kernel(x_cat, tables, col_embed):
    F, V, D = tables.shape
    B = x_cat.shape[0]
    tab_t = tables.transpose(0, 2, 1)
    x_t = x_cat.astype(jnp.int32).T
    out_t = _build(B, F, V, D)(tab_t, x_t, col_embed)
    return out_t.transpose(2, 0, 1)
